# baseline (speedup 1.0000x reference)
import functools

import jax
import jax.numpy as jnp
from jax import lax
from jax.experimental import pallas as pl
from jax.experimental.pallas import tpu as pltpu
from jax.experimental.pallas import tpu_sc as plsc

N_G = 262144
N_M = 128
K_TOP = int(N_G * 0.02)
ROWS = 8192
GRID = N_G // ROWS
OUT_R = N_G // 128

NS = 16
E = N_G // NS
NV = E // 16
NBINS = 1 << 16
CB = NBINS // NS
CVEC = CB // 16
HIST = NBINS + 8
NIDX = 128
NDMA = E // NIDX


def _matvec_body(x_ref, w_ref, m_ref, o_ref):
    mb = (w_ref[...] * m_ref[...]).astype(jnp.bfloat16).astype(jnp.float32)
    xb = x_ref[...].astype(jnp.bfloat16).astype(jnp.float32)
    g = jnp.sum(mb * xb, axis=1)
    o_ref[...] = g.reshape(ROWS // 128, 128)


def _keys(x):
    bits = lax.bitcast_convert_type(x, jnp.int32)
    flip = jnp.where(bits < 0, jnp.uint32(0xFFFFFFFF), jnp.uint32(0x80000000))
    return bits.astype(jnp.uint32) ^ flip


def _tree_sum(v):
    iota = lax.iota(jnp.int32, 16)
    for sh in (8, 4, 2, 1):
        v = v + jnp.take(v, iota ^ sh)
    return v


def _bcast_lane0(v):
    return jnp.take(v, jnp.zeros((16,), jnp.int32))


def _scan_chunk(chunk_ref, base_bin, running0, k_target):
    iota = lax.iota(jnp.int32, 16)
    zero = jnp.zeros((16,), jnp.int32)

    def vec_body(i, carry):
        running, vi_hit, run_hit, vhit, found = carry
        vi = CVEC - 1 - i
        v = chunk_ref[pl.ds(vi * 16, 16)]
        vec_total = _tree_sum(v)
        hit = jnp.logical_and(found == 0, running + vec_total >= k_target)
        vi_hit = jnp.where(hit, vi, vi_hit)
        run_hit = jnp.where(hit, running, run_hit)
        vhit = jnp.where(hit, v, vhit)
        found = jnp.where(hit, 1, found)
        return running + vec_total, vi_hit, run_hit, vhit, found

    init = (running0, zero, running0, zero, zero)
    _, vi_hit, run_hit, vhit, found1 = lax.fori_loop(0, CVEC, vec_body, init)

    def lane_body(i, carry):
        s, jstar, cnt_sel, found = carry
        l = 15 - i
        s = s + _tree_sum(jnp.where(iota == l, vhit, 0))
        hit = jnp.logical_and(found == 0, run_hit + s >= k_target)
        jstar = jnp.where(hit, l, jstar)
        cnt_sel = jnp.where(hit, run_hit + s, cnt_sel)
        found = jnp.where(hit, 1, found)
        return s, jstar, cnt_sel, found

    init2 = (zero, zero, zero, zero)
    _, jstar, cnt_sel, _ = lax.fori_loop(0, 16, lane_body, init2)
    bin_cnt = _tree_sum(jnp.where(iota == jstar, vhit, 0))
    b = base_bin + vi_hit * 16 + jstar
    return b, cnt_sel - bin_cnt, found1


def _select_body(g_hbm, thr_hbm, out_hbm, gv, keyv, outv, idxv, onesv,
                 chunkv, zerov, v16a, v16b, idx16v, thrv, hist_sh, sums1_sh,
                 sums2_sh, bk1_sh, bk2_sh, dma_sem):
    cid = lax.axis_index("c")
    wid = lax.axis_index("s")
    iota = lax.iota(jnp.int32, 16)

    @pl.when(cid == 0)
    def _():
        base = wid * E
        pltpu.sync_copy(g_hbm.at[pl.ds(base, E)], gv)
        pltpu.sync_copy(thr_hbm, thrv)

        def fill_const(i, _):
            zerov[pl.ds(i * 16, 16)] = jnp.zeros((16,), jnp.int32)
            return 0

        lax.fori_loop(0, CVEC, fill_const, 0)
        for j in range(NIDX // 16):
            onesv[pl.ds(j * 16, 16)] = jnp.ones((16,), jnp.int32)
        idx16v[...] = iota

        pltpu.sync_copy(zerov, hist_sh.at[pl.ds(wid * CB, CB)])

        @pl.when(wid == 0)
        def _():
            pltpu.sync_copy(zerov.at[pl.ds(0, 16)], sums1_sh)
            pltpu.sync_copy(zerov.at[pl.ds(0, 16)], sums2_sh)
            pltpu.sync_copy(zerov.at[pl.ds(0, 16)], bk1_sh)
            pltpu.sync_copy(zerov.at[pl.ds(0, 16)], bk2_sh)

        plsc.subcore_barrier()

        def key_bucket(v, _):
            x = gv[pl.ds(v * 16, 16)]
            key = _keys(x)
            keyv[pl.ds(v * 16, 16)] = lax.bitcast_convert_type(key, jnp.int32)
            bucket = (key >> jnp.uint32(16)).astype(jnp.int32)
            r = v // 8
            c = (v % 8) * 16
            idxv[r, pl.ds(c, 16)] = bucket
            return 0

        lax.fori_loop(0, NV, key_bucket, 0)

        def fire_add(j, _):
            pltpu.async_copy(onesv, hist_sh.at[idxv.at[j]], dma_sem,
                             add=True)
            return 0

        def drain(j, _):
            pltpu.make_async_copy(onesv, hist_sh.at[idxv.at[j]],
                                  dma_sem).wait()
            return 0

        lax.fori_loop(0, NDMA, fire_add, 0)
        lax.fori_loop(0, NDMA, drain, 0)
        plsc.subcore_barrier()

        pltpu.sync_copy(hist_sh.at[pl.ds(wid * CB, CB)], chunkv)

        def accum(i, acc):
            return acc + chunkv[pl.ds(i * 16, 16)]

        s_mine = _tree_sum(lax.fori_loop(0, CVEC, accum,
                                         jnp.zeros((16,), jnp.int32)))
        v16a[...] = jnp.where(iota == wid, s_mine, 0)
        pltpu.sync_copy(v16a, sums1_sh.at[idx16v], add=True)
        plsc.subcore_barrier()

        pltpu.sync_copy(sums1_sh, v16b)
        s_vec = v16b[...]
        my_se = _tree_sum(jnp.where(iota > wid, s_vec, 0))
        k1 = jnp.full((16,), K_TOP, jnp.int32)
        b_cand, cnt_hi, found = _scan_chunk(chunkv, wid * CB, my_se, k1)
        is_owner = jnp.logical_and(found > 0, my_se < k1)
        kp_cand = k1 - cnt_hi
        v16a[...] = (jnp.where(jnp.logical_and(is_owner, iota == 0),
                               b_cand, 0)
                     + jnp.where(jnp.logical_and(is_owner, iota == 1),
                                 kp_cand, 0))
        pltpu.sync_copy(v16a, bk1_sh.at[idx16v], add=True)

        pltpu.sync_copy(zerov, hist_sh.at[pl.ds(wid * CB, CB)])
        plsc.subcore_barrier()

        pltpu.sync_copy(bk1_sh, v16b)
        bk = v16b[...]
        b1 = _bcast_lane0(bk)
        kp = _bcast_lane0(jnp.take(bk, jnp.ones((16,), jnp.int32)))

        def low_bucket(v, _):
            key = lax.bitcast_convert_type(keyv[pl.ds(v * 16, 16)],
                                           jnp.uint32)
            hi = (key >> jnp.uint32(16)).astype(jnp.int32)
            lo = (key & jnp.uint32(0xFFFF)).astype(jnp.int32)
            bucket = jnp.where(hi == b1, lo, jnp.int32(NBINS))
            r = v // 8
            c = (v % 8) * 16
            idxv[r, pl.ds(c, 16)] = bucket
            return 0

        lax.fori_loop(0, NV, low_bucket, 0)
        lax.fori_loop(0, NDMA, fire_add, 0)
        lax.fori_loop(0, NDMA, drain, 0)
        plsc.subcore_barrier()

        pltpu.sync_copy(hist_sh.at[pl.ds(wid * CB, CB)], chunkv)
        s_mine2 = _tree_sum(lax.fori_loop(0, CVEC, accum,
                                          jnp.zeros((16,), jnp.int32)))
        v16a[...] = jnp.where(iota == wid, s_mine2, 0)
        pltpu.sync_copy(v16a, sums2_sh.at[idx16v], add=True)
        plsc.subcore_barrier()

        pltpu.sync_copy(sums2_sh, v16b)
        s_vec2 = v16b[...]
        my_se2 = _tree_sum(jnp.where(iota > wid, s_vec2, 0))
        l_cand, _cnt, found2 = _scan_chunk(chunkv, wid * CB, my_se2, kp)
        is_owner2 = jnp.logical_and(found2 > 0, my_se2 < kp)
        t_cand = lax.bitcast_convert_type(
            (lax.bitcast_convert_type(b1, jnp.uint32) << jnp.uint32(16))
            | lax.bitcast_convert_type(l_cand, jnp.uint32), jnp.int32)
        v16a[...] = jnp.where(jnp.logical_and(is_owner2, iota == 0),
                              t_cand, 0)
        pltpu.sync_copy(v16a, bk2_sh.at[idx16v], add=True)
        plsc.subcore_barrier()

        pltpu.sync_copy(bk2_sh, v16b)
        t_u32 = lax.bitcast_convert_type(_bcast_lane0(v16b[...]), jnp.uint32)

        th = thrv[...]

        def write(v, _):
            x = gv[pl.ds(v * 16, 16)]
            key = lax.bitcast_convert_type(keyv[pl.ds(v * 16, 16)],
                                           jnp.uint32)
            keep = key >= t_u32
            outv[pl.ds(v * 16, 16)] = jnp.where(
                keep, jnp.maximum(x - th, 0.0), 0.0)
            return 0

        lax.fori_loop(0, NV, write, 0)
        pltpu.sync_copy(outv, out_hbm.at[pl.ds(base, E)])


def _build():
    matvec = pl.pallas_call(
        _matvec_body,
        grid=(GRID,),
        in_specs=[
            pl.BlockSpec((1, N_M), lambda i: (0, 0)),
            pl.BlockSpec((ROWS, N_M), lambda i: (i, 0)),
            pl.BlockSpec((ROWS, N_M), lambda i: (i, 0)),
        ],
        out_specs=pl.BlockSpec((ROWS // 128, 128), lambda i: (i, 0)),
        out_shape=jax.ShapeDtypeStruct((OUT_R, 128), jnp.float32),
    )
    mesh = plsc.VectorSubcoreMesh(core_axis_name="c", subcore_axis_name="s")
    select = pl.kernel(
        _select_body,
        out_type=jax.ShapeDtypeStruct((N_G,), jnp.float32),
        mesh=mesh,
        scratch_types=[
            pltpu.VMEM((E,), jnp.float32),
            pltpu.VMEM((E,), jnp.int32),
            pltpu.VMEM((E,), jnp.float32),
            pltpu.VMEM((NDMA, NIDX), jnp.int32),
            pltpu.VMEM((NIDX,), jnp.int32),
            pltpu.VMEM((CB,), jnp.int32),
            pltpu.VMEM((CB,), jnp.int32),
            pltpu.VMEM((16,), jnp.int32),
            pltpu.VMEM((16,), jnp.int32),
            pltpu.VMEM((16,), jnp.int32),
            pltpu.VMEM((16,), jnp.float32),
            pltpu.VMEM_SHARED((HIST,), jnp.int32),
            pltpu.VMEM_SHARED((16,), jnp.int32),
            pltpu.VMEM_SHARED((16,), jnp.int32),
            pltpu.VMEM_SHARED((16,), jnp.int32),
            pltpu.VMEM_SHARED((16,), jnp.int32),
            pltpu.SemaphoreType.DMA,
        ],
    )
    return matvec, select


_matvec, _select = _build()


def kernel(mossy_input, weights, connectivity_mask, threshold):
    x = mossy_input.reshape(1, N_M)
    g = _matvec(x, weights, connectivity_mask)
    thr16 = jnp.full((16,), threshold, jnp.float32)
    out = _select(g.reshape(N_G), thr16)
    return out.reshape(N_G)


# speedup vs baseline: 1.8494x; 1.8494x over previous
import functools

import jax
import jax.numpy as jnp
from jax import lax
from jax.experimental import pallas as pl
from jax.experimental.pallas import tpu as pltpu
from jax.experimental.pallas import tpu_sc as plsc

N_G = 262144
N_M = 128
K_TOP = int(N_G * 0.02)
ROWS = 8192
GRID = N_G // ROWS
OUT_R = N_G // 128

NS = 16
E = N_G // NS
NV = E // 16
NBINS = 1 << 16
CB = NBINS // NS
CVEC = CB // 16
HIST = NBINS + 8
NIDX = 128
NDMA = E // NIDX
NREP = 8
HIST_TOT = NREP * HIST


def _matvec_body(x_ref, w_ref, m_ref, o_ref):
    mb = (w_ref[...] * m_ref[...]).astype(jnp.bfloat16).astype(jnp.float32)
    xb = x_ref[...].astype(jnp.bfloat16).astype(jnp.float32)
    g = jnp.sum(mb * xb, axis=1)
    o_ref[...] = g.reshape(ROWS // 128, 128)


def _keys(x):
    bits = lax.bitcast_convert_type(x, jnp.int32)
    flip = jnp.where(bits < 0, jnp.uint32(0xFFFFFFFF), jnp.uint32(0x80000000))
    return bits.astype(jnp.uint32) ^ flip


def _tree_sum(v):
    iota = lax.iota(jnp.int32, 16)
    for sh in (8, 4, 2, 1):
        v = v + jnp.take(v, iota ^ sh)
    return v


def _bcast_lane0(v):
    return jnp.take(v, jnp.zeros((16,), jnp.int32))


def _scan_chunk(chunk_ref, base_bin, running0, k_target):
    iota = lax.iota(jnp.int32, 16)
    zero = jnp.zeros((16,), jnp.int32)

    def vec_body(i, carry):
        running, vi_hit, run_hit, vhit, found = carry
        vi = CVEC - 1 - i
        v = chunk_ref[pl.ds(vi * 16, 16)]
        vec_total = _tree_sum(v)
        hit = jnp.logical_and(found == 0, running + vec_total >= k_target)
        vi_hit = jnp.where(hit, vi, vi_hit)
        run_hit = jnp.where(hit, running, run_hit)
        vhit = jnp.where(hit, v, vhit)
        found = jnp.where(hit, 1, found)
        return running + vec_total, vi_hit, run_hit, vhit, found

    init = (running0, zero, running0, zero, zero)
    _, vi_hit, run_hit, vhit, found1 = lax.fori_loop(0, CVEC, vec_body, init)

    def lane_body(i, carry):
        s, jstar, cnt_sel, found = carry
        l = 15 - i
        s = s + _tree_sum(jnp.where(iota == l, vhit, 0))
        hit = jnp.logical_and(found == 0, run_hit + s >= k_target)
        jstar = jnp.where(hit, l, jstar)
        cnt_sel = jnp.where(hit, run_hit + s, cnt_sel)
        found = jnp.where(hit, 1, found)
        return s, jstar, cnt_sel, found

    init2 = (zero, zero, zero, zero)
    _, jstar, cnt_sel, _ = lax.fori_loop(0, 16, lane_body, init2)
    bin_cnt = _tree_sum(jnp.where(iota == jstar, vhit, 0))
    b = base_bin + vi_hit * 16 + jstar
    return b, cnt_sel - bin_cnt, found1


def _select_body(g_hbm, thr_hbm, out_hbm, gv, idxv, onesv,
                 chunkv, tmpv, zerov, v16a, v16b, idx16v, thrv, hist_sh,
                 sums1_sh, sums2_sh, bk1_sh, bk2_sh, dma_sem):
    cid = lax.axis_index("c")
    wid = lax.axis_index("s")
    iota = lax.iota(jnp.int32, 16)

    @pl.when(cid == 0)
    def _():
        base = wid * E
        pltpu.sync_copy(g_hbm.at[pl.ds(base, E)], gv)
        pltpu.sync_copy(thr_hbm, thrv)

        @plsc.parallel_loop(0, CVEC, unroll=8)
        def _(i):
            zerov[pl.ds(i * 16, 16)] = jnp.zeros((16,), jnp.int32)

        for j in range(NIDX // 16):
            onesv[pl.ds(j * 16, 16)] = jnp.ones((16,), jnp.int32)
        idx16v[...] = iota

        @pl.when(wid < NREP)
        def _():
            for r in range(NS):
                pltpu.sync_copy(zerov,
                                hist_sh.at[pl.ds(wid * HIST + r * CB, CB)])

        @pl.when(wid == 0)
        def _():
            pltpu.sync_copy(zerov.at[pl.ds(0, 16)], sums1_sh)
            pltpu.sync_copy(zerov.at[pl.ds(0, 16)], sums2_sh)
            pltpu.sync_copy(zerov.at[pl.ds(0, 16)], bk1_sh)
            pltpu.sync_copy(zerov.at[pl.ds(0, 16)], bk2_sh)

        plsc.subcore_barrier()

        hist_base = (wid % NREP) * HIST

        @plsc.parallel_loop(0, NV, unroll=8)
        def _(v):
            key = _keys(gv[pl.ds(v * 16, 16)])
            bucket = (key >> jnp.uint32(16)).astype(jnp.int32) + hist_base
            r = v // 8
            c = (v % 8) * 16
            idxv[r, pl.ds(c, 16)] = bucket

        def fire_add(j, _):
            pltpu.async_copy(onesv, hist_sh.at[idxv.at[j]], dma_sem,
                             add=True)
            return 0

        def drain(j, _):
            pltpu.make_async_copy(onesv, hist_sh.at[idxv.at[j]],
                                  dma_sem).wait()
            return 0

        lax.fori_loop(0, NDMA, fire_add, 0)
        lax.fori_loop(0, NDMA, drain, 0)
        plsc.subcore_barrier()

        def merge_chunk():
            pltpu.sync_copy(hist_sh.at[pl.ds(wid * CB, CB)], chunkv)

            def merge_r(r, _):
                pltpu.sync_copy(hist_sh.at[pl.ds(r * HIST + wid * CB, CB)],
                                tmpv)

                @plsc.parallel_loop(0, CVEC, unroll=8)
                def _(i):
                    chunkv[pl.ds(i * 16, 16)] = (chunkv[pl.ds(i * 16, 16)]
                                                 + tmpv[pl.ds(i * 16, 16)])

                return 0

            lax.fori_loop(1, NREP, merge_r, 0)

        merge_chunk()

        def accum(i, acc):
            return acc + chunkv[pl.ds(i * 16, 16)]

        s_mine = _tree_sum(lax.fori_loop(0, CVEC, accum,
                                         jnp.zeros((16,), jnp.int32)))
        v16a[...] = jnp.where(iota == wid, s_mine, 0)
        pltpu.sync_copy(v16a, sums1_sh.at[idx16v], add=True)
        plsc.subcore_barrier()

        pltpu.sync_copy(sums1_sh, v16b)
        s_vec = v16b[...]
        my_se = _tree_sum(jnp.where(iota > wid, s_vec, 0))
        k1 = jnp.full((16,), K_TOP, jnp.int32)
        b_cand, cnt_hi, found = _scan_chunk(chunkv, wid * CB, my_se, k1)
        is_owner = jnp.logical_and(found > 0, my_se < k1)
        kp_cand = k1 - cnt_hi
        v16a[...] = (jnp.where(jnp.logical_and(is_owner, iota == 0),
                               b_cand, 0)
                     + jnp.where(jnp.logical_and(is_owner, iota == 1),
                                 kp_cand, 0))
        pltpu.sync_copy(v16a, bk1_sh.at[idx16v], add=True)

        @pl.when(wid < NREP)
        def _():
            for r in range(NS):
                pltpu.sync_copy(zerov,
                                hist_sh.at[pl.ds(wid * HIST + r * CB, CB)])

        plsc.subcore_barrier()

        pltpu.sync_copy(bk1_sh, v16b)
        bk = v16b[...]
        b1 = _bcast_lane0(bk)
        kp = _bcast_lane0(jnp.take(bk, jnp.ones((16,), jnp.int32)))

        @plsc.parallel_loop(0, NV, unroll=8)
        def _(v):
            key = _keys(gv[pl.ds(v * 16, 16)])
            hi = (key >> jnp.uint32(16)).astype(jnp.int32)
            lo = (key & jnp.uint32(0xFFFF)).astype(jnp.int32)
            bucket = jnp.where(hi == b1, lo, jnp.int32(NBINS)) + hist_base
            r = v // 8
            c = (v % 8) * 16
            idxv[r, pl.ds(c, 16)] = bucket

        lax.fori_loop(0, NDMA, fire_add, 0)
        lax.fori_loop(0, NDMA, drain, 0)
        plsc.subcore_barrier()

        merge_chunk()
        s_mine2 = _tree_sum(lax.fori_loop(0, CVEC, accum,
                                          jnp.zeros((16,), jnp.int32)))
        v16a[...] = jnp.where(iota == wid, s_mine2, 0)
        pltpu.sync_copy(v16a, sums2_sh.at[idx16v], add=True)
        plsc.subcore_barrier()

        pltpu.sync_copy(sums2_sh, v16b)
        s_vec2 = v16b[...]
        my_se2 = _tree_sum(jnp.where(iota > wid, s_vec2, 0))
        l_cand, _cnt, found2 = _scan_chunk(chunkv, wid * CB, my_se2, kp)
        is_owner2 = jnp.logical_and(found2 > 0, my_se2 < kp)
        t_cand = lax.bitcast_convert_type(
            (lax.bitcast_convert_type(b1, jnp.uint32) << jnp.uint32(16))
            | lax.bitcast_convert_type(l_cand, jnp.uint32), jnp.int32)
        v16a[...] = jnp.where(jnp.logical_and(is_owner2, iota == 0),
                              t_cand, 0)
        pltpu.sync_copy(v16a, bk2_sh.at[idx16v], add=True)
        plsc.subcore_barrier()

        pltpu.sync_copy(bk2_sh, v16b)
        t_u32 = lax.bitcast_convert_type(_bcast_lane0(v16b[...]), jnp.uint32)

        th = thrv[...]

        @plsc.parallel_loop(0, NV, unroll=8)
        def _(v):
            x = gv[pl.ds(v * 16, 16)]
            keep = _keys(x) >= t_u32
            gv[pl.ds(v * 16, 16)] = jnp.where(
                keep, jnp.maximum(x - th, 0.0), 0.0)

        pltpu.sync_copy(gv, out_hbm.at[pl.ds(base, E)])


def _build():
    matvec = pl.pallas_call(
        _matvec_body,
        grid=(GRID,),
        in_specs=[
            pl.BlockSpec((1, N_M), lambda i: (0, 0)),
            pl.BlockSpec((ROWS, N_M), lambda i: (i, 0)),
            pl.BlockSpec((ROWS, N_M), lambda i: (i, 0)),
        ],
        out_specs=pl.BlockSpec((ROWS // 128, 128), lambda i: (i, 0)),
        out_shape=jax.ShapeDtypeStruct((OUT_R, 128), jnp.float32),
    )
    mesh = plsc.VectorSubcoreMesh(core_axis_name="c", subcore_axis_name="s")
    select = pl.kernel(
        _select_body,
        out_type=jax.ShapeDtypeStruct((N_G,), jnp.float32),
        mesh=mesh,
        scratch_types=[
            pltpu.VMEM((E,), jnp.float32),
            pltpu.VMEM((NDMA, NIDX), jnp.int32),
            pltpu.VMEM((NIDX,), jnp.int32),
            pltpu.VMEM((CB,), jnp.int32),
            pltpu.VMEM((CB,), jnp.int32),
            pltpu.VMEM((CB,), jnp.int32),
            pltpu.VMEM((16,), jnp.int32),
            pltpu.VMEM((16,), jnp.int32),
            pltpu.VMEM((16,), jnp.int32),
            pltpu.VMEM((16,), jnp.float32),
            pltpu.VMEM_SHARED((HIST_TOT,), jnp.int32),
            pltpu.VMEM_SHARED((16,), jnp.int32),
            pltpu.VMEM_SHARED((16,), jnp.int32),
            pltpu.VMEM_SHARED((16,), jnp.int32),
            pltpu.VMEM_SHARED((16,), jnp.int32),
            pltpu.SemaphoreType.DMA,
        ],
    )
    return matvec, select


_matvec, _select = _build()


def kernel(mossy_input, weights, connectivity_mask, threshold):
    x = mossy_input.reshape(1, N_M)
    g = _matvec(x, weights, connectivity_mask)
    thr16 = jnp.full((16,), threshold, jnp.float32)
    out = _select(g.reshape(N_G), thr16)
    return out.reshape(N_G)


# speedup vs baseline: 1.8825x; 1.0179x over previous
import functools

import jax
import jax.numpy as jnp
from jax import lax
from jax.experimental import pallas as pl
from jax.experimental.pallas import tpu as pltpu
from jax.experimental.pallas import tpu_sc as plsc

N_G = 262144
N_M = 128
K_TOP = int(N_G * 0.02)
ROWS = 8192
GRID = N_G // ROWS
OUT_R = N_G // 128

NS = 16
E = N_G // NS
NV = E // 16
NBINS = 1 << 16
CB = NBINS // NS
CVEC = CB // 16
HIST = NBINS + 8
NIDX = 128
NDMA = E // NIDX
NREP = 8
HIST_TOT = NREP * HIST


def _matvec_body(x_ref, w_ref, m_ref, o_ref):
    mb = (w_ref[...] * m_ref[...]).astype(jnp.bfloat16).astype(jnp.float32)
    xb = x_ref[...].astype(jnp.bfloat16).astype(jnp.float32)
    g = jnp.sum(mb * xb, axis=1)
    o_ref[...] = g.reshape(ROWS // 128, 128)


def _keys(x):
    bits = lax.bitcast_convert_type(x, jnp.int32)
    flip = jnp.where(bits < 0, jnp.uint32(0xFFFFFFFF), jnp.uint32(0x80000000))
    return bits.astype(jnp.uint32) ^ flip


def _tree_sum(v):
    iota = lax.iota(jnp.int32, 16)
    for sh in (8, 4, 2, 1):
        v = v + jnp.take(v, iota ^ sh)
    return v


def _bcast_lane0(v):
    return jnp.take(v, jnp.zeros((16,), jnp.int32))


def _scan_chunk(chunk_ref, base_bin, running0, k_target):
    iota = lax.iota(jnp.int32, 16)
    zero = jnp.zeros((16,), jnp.int32)

    def vec_body(i, carry):
        running, vi_hit, run_hit, vhit, found = carry
        vi = CVEC - 1 - i
        v = chunk_ref[pl.ds(vi * 16, 16)]
        vec_total = _tree_sum(v)
        hit = jnp.logical_and(found == 0, running + vec_total >= k_target)
        vi_hit = jnp.where(hit, vi, vi_hit)
        run_hit = jnp.where(hit, running, run_hit)
        vhit = jnp.where(hit, v, vhit)
        found = jnp.where(hit, 1, found)
        return running + vec_total, vi_hit, run_hit, vhit, found

    init = (running0, zero, running0, zero, zero)
    _, vi_hit, run_hit, vhit, found1 = lax.fori_loop(0, CVEC, vec_body, init)

    def lane_body(i, carry):
        s, jstar, cnt_sel, found = carry
        l = 15 - i
        s = s + _tree_sum(jnp.where(iota == l, vhit, 0))
        hit = jnp.logical_and(found == 0, run_hit + s >= k_target)
        jstar = jnp.where(hit, l, jstar)
        cnt_sel = jnp.where(hit, run_hit + s, cnt_sel)
        found = jnp.where(hit, 1, found)
        return s, jstar, cnt_sel, found

    init2 = (zero, zero, zero, zero)
    _, jstar, cnt_sel, _ = lax.fori_loop(0, 16, lane_body, init2)
    bin_cnt = _tree_sum(jnp.where(iota == jstar, vhit, 0))
    b = base_bin + vi_hit * 16 + jstar
    return b, cnt_sel - bin_cnt, found1


def _select_body(g_hbm, thr_hbm, out_hbm, gv, idxv, onesv,
                 chunkv, tmpv, zerov, v16a, v16b, idx16v, thrv, hist_sh,
                 sums1_sh, sums2_sh, bk1_sh, bk2_sh, dma_sem):
    cid = lax.axis_index("c")
    wid = lax.axis_index("s")
    iota = lax.iota(jnp.int32, 16)

    @pl.when(cid == 0)
    def _():
        base = wid * E
        pltpu.async_copy(g_hbm.at[pl.ds(base, E)], gv, dma_sem)
        pltpu.sync_copy(thr_hbm, thrv)

        @plsc.parallel_loop(0, CVEC, unroll=8)
        def _(i):
            zerov[pl.ds(i * 16, 16)] = jnp.zeros((16,), jnp.int32)

        for j in range(NIDX // 16):
            onesv[pl.ds(j * 16, 16)] = jnp.ones((16,), jnp.int32)
        idx16v[...] = iota

        zbase = (wid % NREP) * HIST + (wid // NREP) * (NBINS // 2)
        for r in range(NS // 2):
            pltpu.sync_copy(zerov, hist_sh.at[pl.ds(zbase + r * CB, CB)])

        @pl.when(wid == 0)
        def _():
            pltpu.sync_copy(zerov.at[pl.ds(0, 16)], sums1_sh)
            pltpu.sync_copy(zerov.at[pl.ds(0, 16)], sums2_sh)
            pltpu.sync_copy(zerov.at[pl.ds(0, 16)], bk1_sh)
            pltpu.sync_copy(zerov.at[pl.ds(0, 16)], bk2_sh)

        pltpu.make_async_copy(g_hbm.at[pl.ds(base, E)], gv, dma_sem).wait()
        plsc.subcore_barrier()

        hist_base = (wid % NREP) * HIST

        @plsc.parallel_loop(0, NV, unroll=8)
        def _(v):
            key = _keys(gv[pl.ds(v * 16, 16)])
            bucket = (key >> jnp.uint32(16)).astype(jnp.int32) + hist_base
            r = v // 8
            c = (v % 8) * 16
            idxv[r, pl.ds(c, 16)] = bucket

        def fire_add(j, _):
            pltpu.async_copy(onesv, hist_sh.at[idxv.at[j]], dma_sem,
                             add=True)
            return 0

        def drain(j, _):
            pltpu.make_async_copy(onesv, hist_sh.at[idxv.at[j]],
                                  dma_sem).wait()
            return 0

        lax.fori_loop(0, NDMA, fire_add, 0)
        lax.fori_loop(0, NDMA, drain, 0)
        plsc.subcore_barrier()

        def merge_chunk():
            pltpu.sync_copy(hist_sh.at[pl.ds(wid * CB, CB)], chunkv)

            def merge_r(r, _):
                pltpu.sync_copy(hist_sh.at[pl.ds(r * HIST + wid * CB, CB)],
                                tmpv)

                @plsc.parallel_loop(0, CVEC, unroll=8)
                def _(i):
                    chunkv[pl.ds(i * 16, 16)] = (chunkv[pl.ds(i * 16, 16)]
                                                 + tmpv[pl.ds(i * 16, 16)])

                return 0

            lax.fori_loop(1, NREP, merge_r, 0)

        merge_chunk()

        def accum(i, acc):
            return acc + chunkv[pl.ds(i * 16, 16)]

        s_mine = _tree_sum(lax.fori_loop(0, CVEC, accum,
                                         jnp.zeros((16,), jnp.int32)))
        v16a[...] = jnp.where(iota == wid, s_mine, 0)
        pltpu.sync_copy(v16a, sums1_sh.at[idx16v], add=True)
        plsc.subcore_barrier()

        pltpu.sync_copy(sums1_sh, v16b)
        s_vec = v16b[...]
        my_se = _tree_sum(jnp.where(iota > wid, s_vec, 0))
        k1 = jnp.full((16,), K_TOP, jnp.int32)
        b_cand, cnt_hi, found = _scan_chunk(chunkv, wid * CB, my_se, k1)
        is_owner = jnp.logical_and(found > 0, my_se < k1)
        kp_cand = k1 - cnt_hi
        v16a[...] = (jnp.where(jnp.logical_and(is_owner, iota == 0),
                               b_cand, 0)
                     + jnp.where(jnp.logical_and(is_owner, iota == 1),
                                 kp_cand, 0))
        pltpu.sync_copy(v16a, bk1_sh.at[idx16v], add=True)

        @pl.when(wid < NREP)
        def _():
            for r in range(NS):
                pltpu.sync_copy(zerov,
                                hist_sh.at[pl.ds(wid * HIST + r * CB, CB)])

        plsc.subcore_barrier()

        pltpu.sync_copy(bk1_sh, v16b)
        bk = v16b[...]
        b1 = _bcast_lane0(bk)
        kp = _bcast_lane0(jnp.take(bk, jnp.ones((16,), jnp.int32)))

        @plsc.parallel_loop(0, NV, unroll=8)
        def _(v):
            key = _keys(gv[pl.ds(v * 16, 16)])
            hi = (key >> jnp.uint32(16)).astype(jnp.int32)
            lo = (key & jnp.uint32(0xFFFF)).astype(jnp.int32)
            bucket = jnp.where(hi == b1, lo, jnp.int32(NBINS)) + hist_base
            r = v // 8
            c = (v % 8) * 16
            idxv[r, pl.ds(c, 16)] = bucket

        lax.fori_loop(0, NDMA, fire_add, 0)
        lax.fori_loop(0, NDMA, drain, 0)
        plsc.subcore_barrier()

        merge_chunk()
        s_mine2 = _tree_sum(lax.fori_loop(0, CVEC, accum,
                                          jnp.zeros((16,), jnp.int32)))
        v16a[...] = jnp.where(iota == wid, s_mine2, 0)
        pltpu.sync_copy(v16a, sums2_sh.at[idx16v], add=True)
        plsc.subcore_barrier()

        pltpu.sync_copy(sums2_sh, v16b)
        s_vec2 = v16b[...]
        my_se2 = _tree_sum(jnp.where(iota > wid, s_vec2, 0))
        l_cand, _cnt, found2 = _scan_chunk(chunkv, wid * CB, my_se2, kp)
        is_owner2 = jnp.logical_and(found2 > 0, my_se2 < kp)
        t_cand = lax.bitcast_convert_type(
            (lax.bitcast_convert_type(b1, jnp.uint32) << jnp.uint32(16))
            | lax.bitcast_convert_type(l_cand, jnp.uint32), jnp.int32)
        v16a[...] = jnp.where(jnp.logical_and(is_owner2, iota == 0),
                              t_cand, 0)
        pltpu.sync_copy(v16a, bk2_sh.at[idx16v], add=True)
        plsc.subcore_barrier()

        pltpu.sync_copy(bk2_sh, v16b)
        t_u32 = lax.bitcast_convert_type(_bcast_lane0(v16b[...]), jnp.uint32)

        th = thrv[...]

        @plsc.parallel_loop(0, NV, unroll=8)
        def _(v):
            x = gv[pl.ds(v * 16, 16)]
            keep = _keys(x) >= t_u32
            gv[pl.ds(v * 16, 16)] = jnp.where(
                keep, jnp.maximum(x - th, 0.0), 0.0)

        pltpu.sync_copy(gv, out_hbm.at[pl.ds(base, E)])


def _build():
    matvec = pl.pallas_call(
        _matvec_body,
        grid=(GRID,),
        in_specs=[
            pl.BlockSpec((1, N_M), lambda i: (0, 0)),
            pl.BlockSpec((ROWS, N_M), lambda i: (i, 0)),
            pl.BlockSpec((ROWS, N_M), lambda i: (i, 0)),
        ],
        out_specs=pl.BlockSpec((ROWS // 128, 128), lambda i: (i, 0)),
        out_shape=jax.ShapeDtypeStruct((OUT_R, 128), jnp.float32),
    )
    mesh = plsc.VectorSubcoreMesh(core_axis_name="c", subcore_axis_name="s")
    select = pl.kernel(
        _select_body,
        out_type=jax.ShapeDtypeStruct((N_G,), jnp.float32),
        mesh=mesh,
        scratch_types=[
            pltpu.VMEM((E,), jnp.float32),
            pltpu.VMEM((NDMA, NIDX), jnp.int32),
            pltpu.VMEM((NIDX,), jnp.int32),
            pltpu.VMEM((CB,), jnp.int32),
            pltpu.VMEM((CB,), jnp.int32),
            pltpu.VMEM((CB,), jnp.int32),
            pltpu.VMEM((16,), jnp.int32),
            pltpu.VMEM((16,), jnp.int32),
            pltpu.VMEM((16,), jnp.int32),
            pltpu.VMEM((16,), jnp.float32),
            pltpu.VMEM_SHARED((HIST_TOT,), jnp.int32),
            pltpu.VMEM_SHARED((16,), jnp.int32),
            pltpu.VMEM_SHARED((16,), jnp.int32),
            pltpu.VMEM_SHARED((16,), jnp.int32),
            pltpu.VMEM_SHARED((16,), jnp.int32),
            pltpu.SemaphoreType.DMA,
        ],
    )
    return matvec, select


_matvec, _select = _build()


def kernel(mossy_input, weights, connectivity_mask, threshold):
    x = mossy_input.reshape(1, N_M)
    g = _matvec(x, weights, connectivity_mask)
    thr16 = jnp.full((16,), threshold, jnp.float32)
    out = _select(g.reshape(N_G), thr16)
    return out.reshape(N_G)


# speedup vs baseline: 1.8848x; 1.0012x over previous
import functools

import jax
import jax.numpy as jnp
from jax import lax
from jax.experimental import pallas as pl
from jax.experimental.pallas import tpu as pltpu
from jax.experimental.pallas import tpu_sc as plsc

N_G = 262144
N_M = 128
K_TOP = int(N_G * 0.02)
ROWS = 16384
GRID = N_G // ROWS
OUT_R = N_G // 128

NS = 16
E = N_G // NS
NV = E // 16
NBINS = 1 << 16
CB = NBINS // NS
CVEC = CB // 16
HIST = NBINS + 8
NIDX = 128
NDMA = E // NIDX
NREP = 8
HIST_TOT = NREP * HIST


def _matvec_body(x_ref, w_ref, m_ref, o_ref):
    mb = (w_ref[...] * m_ref[...]).astype(jnp.bfloat16).astype(jnp.float32)
    xb = x_ref[...].astype(jnp.bfloat16).astype(jnp.float32)
    g = jnp.sum(mb * xb, axis=1)
    o_ref[...] = g.reshape(ROWS // 128, 128)


def _keys(x):
    bits = lax.bitcast_convert_type(x, jnp.int32)
    flip = jnp.where(bits < 0, jnp.uint32(0xFFFFFFFF), jnp.uint32(0x80000000))
    return bits.astype(jnp.uint32) ^ flip


def _tree_sum(v):
    iota = lax.iota(jnp.int32, 16)
    for sh in (8, 4, 2, 1):
        v = v + jnp.take(v, iota ^ sh)
    return v


def _bcast_lane0(v):
    return jnp.take(v, jnp.zeros((16,), jnp.int32))


def _scan_chunk(chunk_ref, base_bin, running0, k_target):
    iota = lax.iota(jnp.int32, 16)
    zero = jnp.zeros((16,), jnp.int32)

    def vec_body(i, carry):
        running, vi_hit, run_hit, vhit, found = carry
        vi = CVEC - 1 - i
        v = chunk_ref[pl.ds(vi * 16, 16)]
        vec_total = _tree_sum(v)
        hit = jnp.logical_and(found == 0, running + vec_total >= k_target)
        vi_hit = jnp.where(hit, vi, vi_hit)
        run_hit = jnp.where(hit, running, run_hit)
        vhit = jnp.where(hit, v, vhit)
        found = jnp.where(hit, 1, found)
        return running + vec_total, vi_hit, run_hit, vhit, found

    init = (running0, zero, running0, zero, zero)
    _, vi_hit, run_hit, vhit, found1 = lax.fori_loop(0, CVEC, vec_body, init)

    def lane_body(i, carry):
        s, jstar, cnt_sel, found = carry
        l = 15 - i
        s = s + _tree_sum(jnp.where(iota == l, vhit, 0))
        hit = jnp.logical_and(found == 0, run_hit + s >= k_target)
        jstar = jnp.where(hit, l, jstar)
        cnt_sel = jnp.where(hit, run_hit + s, cnt_sel)
        found = jnp.where(hit, 1, found)
        return s, jstar, cnt_sel, found

    init2 = (zero, zero, zero, zero)
    _, jstar, cnt_sel, _ = lax.fori_loop(0, 16, lane_body, init2)
    bin_cnt = _tree_sum(jnp.where(iota == jstar, vhit, 0))
    b = base_bin + vi_hit * 16 + jstar
    return b, cnt_sel - bin_cnt, found1


def _select_body(g_hbm, thr_hbm, out_hbm, gv, idxv, onesv,
                 chunkv, tmpv, zerov, v16a, v16b, idx16v, thrv, hist_sh,
                 sums1_sh, sums2_sh, bk1_sh, bk2_sh, dma_sem):
    cid = lax.axis_index("c")
    wid = lax.axis_index("s")
    iota = lax.iota(jnp.int32, 16)

    @pl.when(cid == 0)
    def _():
        base = wid * E
        pltpu.async_copy(g_hbm.at[pl.ds(base, E)], gv, dma_sem)
        pltpu.sync_copy(thr_hbm, thrv)

        @plsc.parallel_loop(0, CVEC, unroll=8)
        def _(i):
            zerov[pl.ds(i * 16, 16)] = jnp.zeros((16,), jnp.int32)

        for j in range(NIDX // 16):
            onesv[pl.ds(j * 16, 16)] = jnp.ones((16,), jnp.int32)
        idx16v[...] = iota

        zbase = (wid % NREP) * HIST + (wid // NREP) * (NBINS // 2)
        for r in range(NS // 2):
            pltpu.sync_copy(zerov, hist_sh.at[pl.ds(zbase + r * CB, CB)])

        @pl.when(wid == 0)
        def _():
            pltpu.sync_copy(zerov.at[pl.ds(0, 16)], sums1_sh)
            pltpu.sync_copy(zerov.at[pl.ds(0, 16)], sums2_sh)
            pltpu.sync_copy(zerov.at[pl.ds(0, 16)], bk1_sh)
            pltpu.sync_copy(zerov.at[pl.ds(0, 16)], bk2_sh)

        pltpu.make_async_copy(g_hbm.at[pl.ds(base, E)], gv, dma_sem).wait()
        plsc.subcore_barrier()

        hist_base = (wid % NREP) * HIST

        @plsc.parallel_loop(0, NV, unroll=8)
        def _(v):
            key = _keys(gv[pl.ds(v * 16, 16)])
            bucket = (key >> jnp.uint32(16)).astype(jnp.int32) + hist_base
            r = v // 8
            c = (v % 8) * 16
            idxv[r, pl.ds(c, 16)] = bucket

        def fire_add(j, _):
            pltpu.async_copy(onesv, hist_sh.at[idxv.at[j]], dma_sem,
                             add=True)
            return 0

        def drain(j, _):
            pltpu.make_async_copy(onesv, hist_sh.at[idxv.at[j]],
                                  dma_sem).wait()
            return 0

        lax.fori_loop(0, NDMA, fire_add, 0)
        lax.fori_loop(0, NDMA, drain, 0)
        plsc.subcore_barrier()

        def merge_chunk():
            pltpu.sync_copy(hist_sh.at[pl.ds(wid * CB, CB)], chunkv)

            def merge_r(r, _):
                pltpu.sync_copy(hist_sh.at[pl.ds(r * HIST + wid * CB, CB)],
                                tmpv)

                @plsc.parallel_loop(0, CVEC, unroll=8)
                def _(i):
                    chunkv[pl.ds(i * 16, 16)] = (chunkv[pl.ds(i * 16, 16)]
                                                 + tmpv[pl.ds(i * 16, 16)])

                return 0

            lax.fori_loop(1, NREP, merge_r, 0)

        merge_chunk()

        def accum(i, acc):
            return acc + chunkv[pl.ds(i * 16, 16)]

        s_mine = _tree_sum(lax.fori_loop(0, CVEC, accum,
                                         jnp.zeros((16,), jnp.int32)))
        v16a[...] = jnp.where(iota == wid, s_mine, 0)
        pltpu.sync_copy(v16a, sums1_sh.at[idx16v], add=True)
        plsc.subcore_barrier()

        pltpu.sync_copy(sums1_sh, v16b)
        s_vec = v16b[...]
        my_se = _tree_sum(jnp.where(iota > wid, s_vec, 0))
        k1 = jnp.full((16,), K_TOP, jnp.int32)
        b_cand, cnt_hi, found = _scan_chunk(chunkv, wid * CB, my_se, k1)
        is_owner = jnp.logical_and(found > 0, my_se < k1)
        kp_cand = k1 - cnt_hi
        v16a[...] = (jnp.where(jnp.logical_and(is_owner, iota == 0),
                               b_cand, 0)
                     + jnp.where(jnp.logical_and(is_owner, iota == 1),
                                 kp_cand, 0))
        pltpu.sync_copy(v16a, bk1_sh.at[idx16v], add=True)

        for r in range(NS // 2):
            pltpu.sync_copy(zerov, hist_sh.at[pl.ds(zbase + r * CB, CB)])

        plsc.subcore_barrier()

        pltpu.sync_copy(bk1_sh, v16b)
        bk = v16b[...]
        b1 = _bcast_lane0(bk)
        kp = _bcast_lane0(jnp.take(bk, jnp.ones((16,), jnp.int32)))

        @plsc.parallel_loop(0, NV, unroll=8)
        def _(v):
            key = _keys(gv[pl.ds(v * 16, 16)])
            hi = (key >> jnp.uint32(16)).astype(jnp.int32)
            lo = (key & jnp.uint32(0xFFFF)).astype(jnp.int32)
            bucket = jnp.where(hi == b1, lo, jnp.int32(NBINS)) + hist_base
            r = v // 8
            c = (v % 8) * 16
            idxv[r, pl.ds(c, 16)] = bucket

        lax.fori_loop(0, NDMA, fire_add, 0)
        lax.fori_loop(0, NDMA, drain, 0)
        plsc.subcore_barrier()

        merge_chunk()
        s_mine2 = _tree_sum(lax.fori_loop(0, CVEC, accum,
                                          jnp.zeros((16,), jnp.int32)))
        v16a[...] = jnp.where(iota == wid, s_mine2, 0)
        pltpu.sync_copy(v16a, sums2_sh.at[idx16v], add=True)
        plsc.subcore_barrier()

        pltpu.sync_copy(sums2_sh, v16b)
        s_vec2 = v16b[...]
        my_se2 = _tree_sum(jnp.where(iota > wid, s_vec2, 0))
        l_cand, _cnt, found2 = _scan_chunk(chunkv, wid * CB, my_se2, kp)
        is_owner2 = jnp.logical_and(found2 > 0, my_se2 < kp)
        t_cand = lax.bitcast_convert_type(
            (lax.bitcast_convert_type(b1, jnp.uint32) << jnp.uint32(16))
            | lax.bitcast_convert_type(l_cand, jnp.uint32), jnp.int32)
        v16a[...] = jnp.where(jnp.logical_and(is_owner2, iota == 0),
                              t_cand, 0)
        pltpu.sync_copy(v16a, bk2_sh.at[idx16v], add=True)
        plsc.subcore_barrier()

        pltpu.sync_copy(bk2_sh, v16b)
        t_u32 = lax.bitcast_convert_type(_bcast_lane0(v16b[...]), jnp.uint32)

        th = thrv[...]

        @plsc.parallel_loop(0, NV, unroll=8)
        def _(v):
            x = gv[pl.ds(v * 16, 16)]
            keep = _keys(x) >= t_u32
            gv[pl.ds(v * 16, 16)] = jnp.where(
                keep, jnp.maximum(x - th, 0.0), 0.0)

        pltpu.sync_copy(gv, out_hbm.at[pl.ds(base, E)])


def _build():
    matvec = pl.pallas_call(
        _matvec_body,
        grid=(GRID,),
        in_specs=[
            pl.BlockSpec((1, N_M), lambda i: (0, 0)),
            pl.BlockSpec((ROWS, N_M), lambda i: (i, 0)),
            pl.BlockSpec((ROWS, N_M), lambda i: (i, 0)),
        ],
        out_specs=pl.BlockSpec((ROWS // 128, 128), lambda i: (i, 0)),
        out_shape=jax.ShapeDtypeStruct((OUT_R, 128), jnp.float32),
    )
    mesh = plsc.VectorSubcoreMesh(core_axis_name="c", subcore_axis_name="s")
    select = pl.kernel(
        _select_body,
        out_type=jax.ShapeDtypeStruct((N_G,), jnp.float32),
        mesh=mesh,
        scratch_types=[
            pltpu.VMEM((E,), jnp.float32),
            pltpu.VMEM((NDMA, NIDX), jnp.int32),
            pltpu.VMEM((NIDX,), jnp.int32),
            pltpu.VMEM((CB,), jnp.int32),
            pltpu.VMEM((CB,), jnp.int32),
            pltpu.VMEM((CB,), jnp.int32),
            pltpu.VMEM((16,), jnp.int32),
            pltpu.VMEM((16,), jnp.int32),
            pltpu.VMEM((16,), jnp.int32),
            pltpu.VMEM((16,), jnp.float32),
            pltpu.VMEM_SHARED((HIST_TOT,), jnp.int32),
            pltpu.VMEM_SHARED((16,), jnp.int32),
            pltpu.VMEM_SHARED((16,), jnp.int32),
            pltpu.VMEM_SHARED((16,), jnp.int32),
            pltpu.VMEM_SHARED((16,), jnp.int32),
            pltpu.SemaphoreType.DMA,
        ],
    )
    return matvec, select


_matvec, _select = _build()


def kernel(mossy_input, weights, connectivity_mask, threshold):
    x = mossy_input.reshape(1, N_M)
    g = _matvec(x, weights, connectivity_mask)
    thr16 = jnp.full((16,), threshold, jnp.float32)
    out = _select(g.reshape(N_G), thr16)
    return out.reshape(N_G)
